# baseline (speedup 1.0000x reference)
import functools

import jax
import jax.numpy as jnp
from jax import lax
from jax.experimental import pallas as pl
from jax.experimental.pallas import tpu as pltpu
from jax.experimental.pallas import tpu_sc as plsc

NC = 2
NS = 16
CHUNK = 80
ZROWS = 40
OUT_TILES = 10


def _sc_segsum(N, D, E, gather):
  E_core = E // NC
  per_tile = E_core // NS
  n_chunks = per_tile // CHUNK
  rows_per_out_tile = N // OUT_TILES

  mesh = plsc.VectorSubcoreMesh(core_axis_name="c", subcore_axis_name="s")

  scratch = [
      pltpu.VMEM((n_chunks, CHUNK), jnp.int32),
      pltpu.VMEM((CHUNK, D), jnp.float32),
      pltpu.VMEM((CHUNK, D), jnp.float32),
      pltpu.VMEM_SHARED((N, D), jnp.float32),
      pltpu.SemaphoreType.DMA,
      pltpu.SemaphoreType.DMA,
      pltpu.SemaphoreType.DMA,
      pltpu.SemaphoreType.DMA,
  ]
  if gather:
    scratch.append(pltpu.VMEM((per_tile,), jnp.int32))

  def body(src_hbm, dst_hbm, h_hbm, zero_hbm, out_hbm, *rest):
    if gather:
      dstb, ra, rb, acc_sh, sga, sgb, ssa, ssb, srcb = rest
    else:
      dstb, ra, rb, acc_sh, sga, sgb, ssa, ssb = rest
    c = lax.axis_index("c")
    s = lax.axis_index("s")
    wid = c * NS + s

    pltpu.sync_copy(dst_hbm.at[wid], dstb)
    if gather:
      pltpu.sync_copy(src_hbm.at[pl.ds(wid * per_tile, per_tile)], srcb)
      pltpu.async_copy(h_hbm.at[srcb.at[pl.ds(0, CHUNK)]], ra, sga)

    if not gather:
      ovec = jnp.ones((16,), jnp.float32)
      lanes = D // 16
      def ofill(i, _):
        ra[i // lanes, pl.ds((i % lanes) * 16, 16)] = ovec
        return 0
      lax.fori_loop(0, CHUNK * lanes, ofill, 0)

    so = jnp.minimum(s, OUT_TILES - 1)
    r0 = so * rows_per_out_tile
    pltpu.sync_copy(zero_hbm.at[pl.ds(r0, rows_per_out_tile)],
                    acc_sh.at[pl.ds(r0, rows_per_out_tile)])
    plsc.subcore_barrier()

    def g_start(i, buf, sem):
      pltpu.async_copy(h_hbm.at[srcb.at[pl.ds(i * CHUNK, CHUNK)]], buf, sem)

    def g_wait(i, buf, sem):
      pltpu.make_async_copy(h_hbm.at[srcb.at[pl.ds(i * CHUNK, CHUNK)]],
                            buf, sem).wait()

    def sc_start(i, buf, sem):
      pltpu.async_copy(buf, acc_sh.at[dstb.at[i]], sem, add=True)

    def sc_wait(i, buf, sem):
      pltpu.make_async_copy(buf, acc_sh.at[dstb.at[i]], sem).wait()

    if gather:
      assert n_chunks % 2 == 1 and n_chunks >= 5
      g_wait(0, ra, sga)
      sc_start(0, ra, ssa)
      g_start(1, rb, sgb)

      def step2(k, _):
        i = 2 * k + 1
        g_wait(i, rb, sgb)
        sc_start(i, rb, ssb)
        sc_wait(i - 1, ra, ssa)
        g_start(i + 1, ra, sga)
        g_wait(i + 1, ra, sga)
        sc_start(i + 1, ra, ssa)
        sc_wait(i, rb, ssb)
        g_start(i + 2, rb, sgb)
        return 0
      lax.fori_loop(0, (n_chunks - 3) // 2, step2, 0)

      i = n_chunks - 2
      g_wait(i, rb, sgb)
      sc_start(i, rb, ssb)
      sc_wait(i - 1, ra, ssa)
      g_start(i + 1, ra, sga)
      g_wait(i + 1, ra, sga)
      sc_start(i + 1, ra, ssa)
      sc_wait(i, rb, ssb)
      sc_wait(i + 1, ra, ssa)
    else:
      assert n_chunks % 2 == 1
      def step2c(k, _):
        i = 2 * k
        sc_start(i, ra, ssa)
        sc_start(i + 1, ra, ssb)
        sc_wait(i, ra, ssa)
        sc_wait(i + 1, ra, ssb)
        return 0
      lax.fori_loop(0, (n_chunks - 1) // 2, step2c, 0)
      sc_start(n_chunks - 1, ra, ssa)
      sc_wait(n_chunks - 1, ra, ssa)

    plsc.subcore_barrier()

    r0 = so * rows_per_out_tile
    pltpu.sync_copy(acc_sh.at[pl.ds(r0, rows_per_out_tile)],
                    out_hbm.at[c, pl.ds(r0, rows_per_out_tile)])

  return pl.kernel(body,
                   out_type=jax.ShapeDtypeStruct((NC, N, D), jnp.float32),
                   mesh=mesh, scratch_types=scratch,
                   name=f"sc_segsum_g{int(gather)}")


def _dot_t(x, w):
  return lax.dot_general(x, w, (((1,), (1,)), ((), ())),
                         preferred_element_type=jnp.float32)


def _tc_pre(x, wl, wr, block):
  N, D = x.shape
  grid = N // block

  def body(x_ref, wl_ref, wr_ref, h_ref, xr_ref):
    xv = x_ref[...]
    h_ref[...] = _dot_t(xv, wl_ref[...])
    xr_ref[...] = _dot_t(xv, wr_ref[...])

  return pl.pallas_call(
      body,
      grid=(grid,),
      in_specs=[
          pl.BlockSpec((block, D), lambda i: (i, 0)),
          pl.BlockSpec((D, D), lambda i: (0, 0)),
          pl.BlockSpec((D, D), lambda i: (0, 0)),
      ],
      out_specs=[
          pl.BlockSpec((block, D), lambda i: (i, 0)),
          pl.BlockSpec((block, D), lambda i: (i, 0)),
      ],
      out_shape=[jax.ShapeDtypeStruct((N, D), jnp.float32)] * 2,
      name="tc_pre",
  )(x, wl, wr)


def _tc_mid(s, cnt, xr, b, wl, wr, block):
  _, N, D = s.shape
  grid = N // block

  def body(s_ref, cnt_ref, xr_ref, b_ref, wl_ref, wr_ref, h_ref, xrn_ref):
    ssum = s_ref[0] + s_ref[1]
    csum = cnt_ref[0, :, 0:1] + cnt_ref[1, :, 0:1]
    inv = 1.0 / jnp.maximum(csum, 1.0)
    xv = jnp.maximum(ssum * inv + b_ref[...] + xr_ref[...], 0.0)
    h_ref[...] = _dot_t(xv, wl_ref[...])
    xrn_ref[...] = _dot_t(xv, wr_ref[...])

  return pl.pallas_call(
      body,
      grid=(grid,),
      in_specs=[
          pl.BlockSpec((NC, block, D), lambda i: (0, i, 0)),
          pl.BlockSpec((NC, block, cnt.shape[2]), lambda i: (0, i, 0)),
          pl.BlockSpec((block, D), lambda i: (i, 0)),
          pl.BlockSpec((1, D), lambda i: (0, 0)),
          pl.BlockSpec((D, D), lambda i: (0, 0)),
          pl.BlockSpec((D, D), lambda i: (0, 0)),
      ],
      out_specs=[
          pl.BlockSpec((block, D), lambda i: (i, 0)),
          pl.BlockSpec((block, D), lambda i: (i, 0)),
      ],
      out_shape=[jax.ShapeDtypeStruct((N, D), jnp.float32)] * 2,
      name="tc_mid",
  )(s, cnt, xr, b, wl, wr)


def _tc_post(s, cnt, xr, b, block):
  _, N, D = s.shape
  grid = N // block

  def body(s_ref, cnt_ref, xr_ref, b_ref, o_ref):
    ssum = s_ref[0] + s_ref[1]
    csum = cnt_ref[0, :, 0:1] + cnt_ref[1, :, 0:1]
    inv = 1.0 / jnp.maximum(csum, 1.0)
    o_ref[...] = ssum * inv + b_ref[...] + xr_ref[...]

  return pl.pallas_call(
      body,
      grid=(grid,),
      in_specs=[
          pl.BlockSpec((NC, block, D), lambda i: (0, i, 0)),
          pl.BlockSpec((NC, block, cnt.shape[2]), lambda i: (0, i, 0)),
          pl.BlockSpec((block, D), lambda i: (i, 0)),
          pl.BlockSpec((1, D), lambda i: (0, 0)),
      ],
      out_specs=pl.BlockSpec((block, D), lambda i: (i, 0)),
      out_shape=jax.ShapeDtypeStruct((N, D), jnp.float32),
      name="tc_post",
  )(s, cnt, xr, b)


@jax.jit
def _run(edge_index, x_all, W1l, b1l, W1r, W2l, b2l, W2r, W3l, b3l, W3r):
  N, D = x_all.shape
  E = edge_index.shape[1]
  block = 2000

  NW = NC * NS
  nch = E // NW // CHUNK
  src = edge_index[0].astype(jnp.int32)
  dst = edge_index[1].astype(jnp.int32).reshape(NW, nch, CHUNK)
  zeros = jnp.zeros((N, D), jnp.float32)
  b1 = b1l.reshape(1, D)
  b2 = b2l.reshape(1, D)
  b3 = b3l.reshape(1, D)

  seg = _sc_segsum(N, D, E, gather=True)
  CW = 128
  seg_cnt = _sc_segsum(N, CW, E, gather=False)
  zeros_c = jnp.zeros((N, CW), jnp.float32)

  cnt = seg_cnt(src, dst, zeros_c, zeros_c)
  h1, xr1 = _tc_pre(x_all, W1l, W1r, block)
  s1 = seg(src, dst, h1, zeros)
  h2, xr2 = _tc_mid(s1, cnt, xr1, b1, W2l, W2r, block)
  s2 = seg(src, dst, h2, zeros)
  h3, xr3 = _tc_mid(s2, cnt, xr2, b2, W3l, W3r, block)
  s3 = seg(src, dst, h3, zeros)
  return _tc_post(s3, cnt, xr3, b3, block)


def kernel(edge_index, x_all, W1l, b1l, W1r, W2l, b2l, W2r, W3l, b3l, W3r):
  return _run(edge_index, x_all, W1l, b1l, W1r, W2l, b2l, W2r, W3l, b3l, W3r)


# speedup vs baseline: 1.0016x; 1.0016x over previous
import jax
import jax.numpy as jnp
from jax import lax
from jax.experimental import pallas as pl
from jax.experimental.pallas import tpu as pltpu
from jax.experimental.pallas import tpu_sc as plsc

NC = 2
NS = 16
CHUNK = 80
OUT_TILES = 10


def _sc_segsum(N, D, E, gather):
  E_core = E // NC
  per_tile = E_core // NS
  n_chunks = per_tile // CHUNK
  rows_per_out_tile = N // OUT_TILES

  mesh = plsc.VectorSubcoreMesh(core_axis_name="c", subcore_axis_name="s")

  scratch = [
      pltpu.VMEM((n_chunks, CHUNK), jnp.int32),
      pltpu.VMEM((CHUNK, D), jnp.float32),
      pltpu.VMEM((CHUNK, D), jnp.float32),
      pltpu.VMEM_SHARED((N, D), jnp.float32),
      pltpu.SemaphoreType.DMA,
      pltpu.SemaphoreType.DMA,
      pltpu.SemaphoreType.DMA,
      pltpu.SemaphoreType.DMA,
  ]
  if gather:
    scratch.append(pltpu.VMEM((per_tile,), jnp.int32))

  def body(src_hbm, dst_hbm, h_hbm, zero_hbm, out_hbm, *rest):
    if gather:
      dstb, ra, rb, acc_sh, sga, sgb, ssa, ssb, srcb = rest
    else:
      dstb, ra, rb, acc_sh, sga, sgb, ssa, ssb = rest
    c = lax.axis_index("c")
    s = lax.axis_index("s")
    wid = c * NS + s

    pltpu.sync_copy(dst_hbm.at[wid], dstb)
    if gather:
      pltpu.sync_copy(src_hbm.at[pl.ds(wid * per_tile, per_tile)], srcb)
      pltpu.async_copy(h_hbm.at[srcb.at[pl.ds(0, CHUNK)]], ra, sga)

    if not gather:
      ovec = jnp.ones((16,), jnp.float32)
      lanes = D // 16
      def ofill(i, _):
        ra[i // lanes, pl.ds((i % lanes) * 16, 16)] = ovec
        return 0
      lax.fori_loop(0, CHUNK * lanes, ofill, 0)

    so = jnp.minimum(s, OUT_TILES - 1)
    r0 = so * rows_per_out_tile
    pltpu.sync_copy(zero_hbm.at[pl.ds(r0, rows_per_out_tile)],
                    acc_sh.at[pl.ds(r0, rows_per_out_tile)])
    plsc.subcore_barrier()

    def g_start(i, buf, sem):
      pltpu.async_copy(h_hbm.at[srcb.at[pl.ds(i * CHUNK, CHUNK)]], buf, sem)

    def g_wait(i, buf, sem):
      pltpu.make_async_copy(h_hbm.at[srcb.at[pl.ds(i * CHUNK, CHUNK)]],
                            buf, sem).wait()

    def sc_start(i, buf, sem):
      pltpu.async_copy(buf, acc_sh.at[dstb.at[i]], sem, add=True)

    def sc_wait(i, buf, sem):
      pltpu.make_async_copy(buf, acc_sh.at[dstb.at[i]], sem).wait()

    if gather:
      assert n_chunks % 2 == 1 and n_chunks >= 5
      g_wait(0, ra, sga)
      sc_start(0, ra, ssa)
      g_start(1, rb, sgb)

      def step2(k, _):
        i = 2 * k + 1
        g_wait(i, rb, sgb)
        sc_start(i, rb, ssb)
        sc_wait(i - 1, ra, ssa)
        g_start(i + 1, ra, sga)
        g_wait(i + 1, ra, sga)
        sc_start(i + 1, ra, ssa)
        sc_wait(i, rb, ssb)
        g_start(i + 2, rb, sgb)
        return 0
      lax.fori_loop(0, (n_chunks - 3) // 2, step2, 0)

      i = n_chunks - 2
      g_wait(i, rb, sgb)
      sc_start(i, rb, ssb)
      sc_wait(i - 1, ra, ssa)
      g_start(i + 1, ra, sga)
      g_wait(i + 1, ra, sga)
      sc_start(i + 1, ra, ssa)
      sc_wait(i, rb, ssb)
      sc_wait(i + 1, ra, ssa)
    else:
      assert n_chunks % 2 == 1
      def step2c(k, _):
        i = 2 * k
        sc_start(i, ra, ssa)
        sc_start(i + 1, ra, ssb)
        sc_wait(i, ra, ssa)
        sc_wait(i + 1, ra, ssb)
        return 0
      lax.fori_loop(0, (n_chunks - 1) // 2, step2c, 0)
      sc_start(n_chunks - 1, ra, ssa)
      sc_wait(n_chunks - 1, ra, ssa)

    plsc.subcore_barrier()

    r0 = so * rows_per_out_tile
    pltpu.sync_copy(acc_sh.at[pl.ds(r0, rows_per_out_tile)],
                    out_hbm.at[c, pl.ds(r0, rows_per_out_tile)])

  return pl.kernel(body,
                   out_type=jax.ShapeDtypeStruct((NC, N, D), jnp.float32),
                   mesh=mesh, scratch_types=scratch,
                   name=f"sc_segsum_g{int(gather)}")


def _dot_t(x, w):
  return lax.dot_general(x, w, (((1,), (1,)), ((), ())),
                         preferred_element_type=jnp.float32)


def _tc_pre(x, wl, wr, block):
  N, D = x.shape
  grid = N // block

  def body(x_ref, wl_ref, wr_ref, h_ref, xr_ref):
    xv = x_ref[...]
    h_ref[...] = _dot_t(xv, wl_ref[...])
    xr_ref[...] = _dot_t(xv, wr_ref[...])

  return pl.pallas_call(
      body,
      grid=(grid,),
      in_specs=[
          pl.BlockSpec((block, D), lambda i: (i, 0)),
          pl.BlockSpec((D, D), lambda i: (0, 0)),
          pl.BlockSpec((D, D), lambda i: (0, 0)),
      ],
      out_specs=[
          pl.BlockSpec((block, D), lambda i: (i, 0)),
          pl.BlockSpec((block, D), lambda i: (i, 0)),
      ],
      out_shape=[jax.ShapeDtypeStruct((N, D), jnp.float32)] * 2,
      name="tc_pre",
  )(x, wl, wr)


def _tc_mid(s, cnt, xr, b, wl, wr, block):
  _, N, D = s.shape
  grid = N // block

  def body(s_ref, cnt_ref, xr_ref, b_ref, wl_ref, wr_ref, h_ref, xrn_ref):
    ssum = s_ref[0] + s_ref[1]
    csum = cnt_ref[0, :, 0:1] + cnt_ref[1, :, 0:1]
    inv = 1.0 / jnp.maximum(csum, 1.0)
    xv = jnp.maximum(ssum * inv + b_ref[...] + xr_ref[...], 0.0)
    h_ref[...] = _dot_t(xv, wl_ref[...])
    xrn_ref[...] = _dot_t(xv, wr_ref[...])

  return pl.pallas_call(
      body,
      grid=(grid,),
      in_specs=[
          pl.BlockSpec((NC, block, D), lambda i: (0, i, 0)),
          pl.BlockSpec((NC, block, cnt.shape[2]), lambda i: (0, i, 0)),
          pl.BlockSpec((block, D), lambda i: (i, 0)),
          pl.BlockSpec((1, D), lambda i: (0, 0)),
          pl.BlockSpec((D, D), lambda i: (0, 0)),
          pl.BlockSpec((D, D), lambda i: (0, 0)),
      ],
      out_specs=[
          pl.BlockSpec((block, D), lambda i: (i, 0)),
          pl.BlockSpec((block, D), lambda i: (i, 0)),
      ],
      out_shape=[jax.ShapeDtypeStruct((N, D), jnp.float32)] * 2,
      name="tc_mid",
  )(s, cnt, xr, b, wl, wr)


def _tc_post(s, cnt, xr, b, block):
  _, N, D = s.shape
  grid = N // block

  def body(s_ref, cnt_ref, xr_ref, b_ref, o_ref):
    ssum = s_ref[0] + s_ref[1]
    csum = cnt_ref[0, :, 0:1] + cnt_ref[1, :, 0:1]
    inv = 1.0 / jnp.maximum(csum, 1.0)
    o_ref[...] = ssum * inv + b_ref[...] + xr_ref[...]

  return pl.pallas_call(
      body,
      grid=(grid,),
      in_specs=[
          pl.BlockSpec((NC, block, D), lambda i: (0, i, 0)),
          pl.BlockSpec((NC, block, cnt.shape[2]), lambda i: (0, i, 0)),
          pl.BlockSpec((block, D), lambda i: (i, 0)),
          pl.BlockSpec((1, D), lambda i: (0, 0)),
      ],
      out_specs=pl.BlockSpec((block, D), lambda i: (i, 0)),
      out_shape=jax.ShapeDtypeStruct((N, D), jnp.float32),
      name="tc_post",
  )(s, cnt, xr, b)


@jax.jit
def _run(edge_index, x_all, W1l, b1l, W1r, W2l, b2l, W2r, W3l, b3l, W3r):
  N, D = x_all.shape
  E = edge_index.shape[1]
  block = 2000

  NW = NC * NS
  nch = E // NW // CHUNK
  src = edge_index[0].astype(jnp.int32)
  dst = edge_index[1].astype(jnp.int32).reshape(NW, nch, CHUNK)
  zeros = jnp.zeros((N, D), jnp.float32)
  b1 = b1l.reshape(1, D)
  b2 = b2l.reshape(1, D)
  b3 = b3l.reshape(1, D)

  seg = _sc_segsum(N, D, E, gather=True)
  CW = 128
  seg_cnt = _sc_segsum(N, CW, E, gather=False)
  zeros_c = jnp.zeros((N, CW), jnp.float32)

  cnt = seg_cnt(src, dst, zeros_c, zeros_c)
  h1, xr1 = _tc_pre(x_all, W1l, W1r, block)
  s1 = seg(src, dst, h1, zeros)
  h2, xr2 = _tc_mid(s1, cnt, xr1, b1, W2l, W2r, block)
  s2 = seg(src, dst, h2, zeros)
  h3, xr3 = _tc_mid(s2, cnt, xr2, b2, W3l, W3r, block)
  s3 = seg(src, dst, h3, zeros)
  return _tc_post(s3, cnt, xr3, b3, block)


def kernel(edge_index, x_all, W1l, b1l, W1r, W2l, b2l, W2r, W3l, b3l, W3r):
  return _run(edge_index, x_all, W1l, b1l, W1r, W2l, b2l, W2r, W3l, b3l, W3r)


# speedup vs baseline: 1.0044x; 1.0028x over previous
import jax
import jax.numpy as jnp
from jax import lax
from jax.experimental import pallas as pl
from jax.experimental.pallas import tpu as pltpu
from jax.experimental.pallas import tpu_sc as plsc

NC = 2
NS = 16
CHUNK = 80
OUT_TILES = 10


def _sc_segsum(N, D, E, gather):
  E_core = E // NC
  per_tile = E_core // NS
  n_chunks = per_tile // CHUNK
  rows_per_out_tile = N // OUT_TILES

  mesh = plsc.VectorSubcoreMesh(core_axis_name="c", subcore_axis_name="s")

  scratch = [
      pltpu.VMEM((per_tile,), jnp.int32),
      pltpu.VMEM((CHUNK, D), jnp.float32),
      pltpu.VMEM((CHUNK, D), jnp.float32),
      pltpu.VMEM_SHARED((N, D), jnp.float32),
      pltpu.SemaphoreType.DMA,
      pltpu.SemaphoreType.DMA,
      pltpu.SemaphoreType.DMA,
      pltpu.SemaphoreType.DMA,
  ]
  if gather:
    scratch.append(pltpu.VMEM((per_tile,), jnp.int32))

  def body(src_hbm, dst_hbm, h_hbm, zero_hbm, out_hbm, *rest):
    if gather:
      dstb, ra, rb, acc_sh, sga, sgb, ssa, ssb, srcb = rest
    else:
      dstb, ra, rb, acc_sh, sga, sgb, ssa, ssb = rest
    c = lax.axis_index("c")
    s = lax.axis_index("s")
    wid = c * NS + s

    pltpu.sync_copy(dst_hbm.at[pl.ds(wid * per_tile, per_tile)], dstb)
    if gather:
      pltpu.sync_copy(src_hbm.at[pl.ds(wid * per_tile, per_tile)], srcb)
      pltpu.async_copy(h_hbm.at[srcb.at[pl.ds(0, CHUNK)]], ra, sga)

    if not gather:
      ovec = jnp.ones((16,), jnp.float32)
      lanes = D // 16
      def ofill(i, _):
        ra[i // lanes, pl.ds((i % lanes) * 16, 16)] = ovec
        return 0
      lax.fori_loop(0, CHUNK * lanes, ofill, 0)

    so = jnp.minimum(s, OUT_TILES - 1)
    r0 = so * rows_per_out_tile
    pltpu.sync_copy(zero_hbm.at[pl.ds(r0, rows_per_out_tile)],
                    acc_sh.at[pl.ds(r0, rows_per_out_tile)])
    plsc.subcore_barrier()

    def g_start(i, buf, sem):
      pltpu.async_copy(h_hbm.at[srcb.at[pl.ds(i * CHUNK, CHUNK)]], buf, sem)

    def g_wait(i, buf, sem):
      pltpu.make_async_copy(h_hbm.at[srcb.at[pl.ds(i * CHUNK, CHUNK)]],
                            buf, sem).wait()

    def sc_start(i, buf, sem):
      pltpu.async_copy(buf, acc_sh.at[dstb.at[pl.ds(i * CHUNK, CHUNK)]],
                       sem, add=True)

    def sc_wait(i, buf, sem):
      pltpu.make_async_copy(buf, acc_sh.at[dstb.at[pl.ds(i * CHUNK, CHUNK)]],
                            sem).wait()

    if gather:
      assert n_chunks % 2 == 1 and n_chunks >= 5
      g_wait(0, ra, sga)
      sc_start(0, ra, ssa)
      g_start(1, rb, sgb)

      def step2(k, _):
        i = 2 * k + 1
        g_wait(i, rb, sgb)
        sc_start(i, rb, ssb)
        sc_wait(i - 1, ra, ssa)
        g_start(i + 1, ra, sga)
        g_wait(i + 1, ra, sga)
        sc_start(i + 1, ra, ssa)
        sc_wait(i, rb, ssb)
        g_start(i + 2, rb, sgb)
        return 0
      lax.fori_loop(0, (n_chunks - 3) // 2, step2, 0)

      i = n_chunks - 2
      g_wait(i, rb, sgb)
      sc_start(i, rb, ssb)
      sc_wait(i - 1, ra, ssa)
      g_start(i + 1, ra, sga)
      g_wait(i + 1, ra, sga)
      sc_start(i + 1, ra, ssa)
      sc_wait(i, rb, ssb)
      sc_wait(i + 1, ra, ssa)
    else:
      assert n_chunks % 2 == 1
      def step2c(k, _):
        i = 2 * k
        sc_start(i, ra, ssa)
        sc_start(i + 1, ra, ssb)
        sc_wait(i, ra, ssa)
        sc_wait(i + 1, ra, ssb)
        return 0
      lax.fori_loop(0, (n_chunks - 1) // 2, step2c, 0)
      sc_start(n_chunks - 1, ra, ssa)
      sc_wait(n_chunks - 1, ra, ssa)

    plsc.subcore_barrier()

    r0 = so * rows_per_out_tile
    pltpu.sync_copy(acc_sh.at[pl.ds(r0, rows_per_out_tile)],
                    out_hbm.at[c, pl.ds(r0, rows_per_out_tile)])

  return pl.kernel(body,
                   out_type=jax.ShapeDtypeStruct((NC, N, D), jnp.float32),
                   mesh=mesh, scratch_types=scratch,
                   name=f"sc_segsum_g{int(gather)}")


def _dot_t(x, w):
  return lax.dot_general(x, w, (((1,), (1,)), ((), ())),
                         preferred_element_type=jnp.float32)


def _tc_pre(x, wl, wr, block):
  N, D = x.shape
  grid = N // block

  def body(x_ref, wl_ref, wr_ref, h_ref, xr_ref):
    xv = x_ref[...]
    h_ref[...] = _dot_t(xv, wl_ref[...])
    xr_ref[...] = _dot_t(xv, wr_ref[...])

  return pl.pallas_call(
      body,
      grid=(grid,),
      in_specs=[
          pl.BlockSpec((block, D), lambda i: (i, 0)),
          pl.BlockSpec((D, D), lambda i: (0, 0)),
          pl.BlockSpec((D, D), lambda i: (0, 0)),
      ],
      out_specs=[
          pl.BlockSpec((block, D), lambda i: (i, 0)),
          pl.BlockSpec((block, D), lambda i: (i, 0)),
      ],
      out_shape=[jax.ShapeDtypeStruct((N, D), jnp.float32)] * 2,
      name="tc_pre",
  )(x, wl, wr)


def _tc_mid(s, cnt, xr, b, wl, wr, block):
  _, N, D = s.shape
  grid = N // block

  def body(s_ref, cnt_ref, xr_ref, b_ref, wl_ref, wr_ref, h_ref, xrn_ref):
    ssum = s_ref[0] + s_ref[1]
    csum = cnt_ref[0, :, 0:1] + cnt_ref[1, :, 0:1]
    inv = 1.0 / jnp.maximum(csum, 1.0)
    xv = jnp.maximum(ssum * inv + b_ref[...] + xr_ref[...], 0.0)
    h_ref[...] = _dot_t(xv, wl_ref[...])
    xrn_ref[...] = _dot_t(xv, wr_ref[...])

  return pl.pallas_call(
      body,
      grid=(grid,),
      in_specs=[
          pl.BlockSpec((NC, block, D), lambda i: (0, i, 0)),
          pl.BlockSpec((NC, block, cnt.shape[2]), lambda i: (0, i, 0)),
          pl.BlockSpec((block, D), lambda i: (i, 0)),
          pl.BlockSpec((1, D), lambda i: (0, 0)),
          pl.BlockSpec((D, D), lambda i: (0, 0)),
          pl.BlockSpec((D, D), lambda i: (0, 0)),
      ],
      out_specs=[
          pl.BlockSpec((block, D), lambda i: (i, 0)),
          pl.BlockSpec((block, D), lambda i: (i, 0)),
      ],
      out_shape=[jax.ShapeDtypeStruct((N, D), jnp.float32)] * 2,
      name="tc_mid",
  )(s, cnt, xr, b, wl, wr)


def _tc_post(s, cnt, xr, b, block):
  _, N, D = s.shape
  grid = N // block

  def body(s_ref, cnt_ref, xr_ref, b_ref, o_ref):
    ssum = s_ref[0] + s_ref[1]
    csum = cnt_ref[0, :, 0:1] + cnt_ref[1, :, 0:1]
    inv = 1.0 / jnp.maximum(csum, 1.0)
    o_ref[...] = ssum * inv + b_ref[...] + xr_ref[...]

  return pl.pallas_call(
      body,
      grid=(grid,),
      in_specs=[
          pl.BlockSpec((NC, block, D), lambda i: (0, i, 0)),
          pl.BlockSpec((NC, block, cnt.shape[2]), lambda i: (0, i, 0)),
          pl.BlockSpec((block, D), lambda i: (i, 0)),
          pl.BlockSpec((1, D), lambda i: (0, 0)),
      ],
      out_specs=pl.BlockSpec((block, D), lambda i: (i, 0)),
      out_shape=jax.ShapeDtypeStruct((N, D), jnp.float32),
      name="tc_post",
  )(s, cnt, xr, b)


@jax.jit
def _run(edge_index, x_all, W1l, b1l, W1r, W2l, b2l, W2r, W3l, b3l, W3r):
  N, D = x_all.shape
  E = edge_index.shape[1]
  block = 2000

  src = edge_index[0].astype(jnp.int32)
  dst = edge_index[1].astype(jnp.int32)
  zeros = jnp.zeros((N, D), jnp.float32)
  b1 = b1l.reshape(1, D)
  b2 = b2l.reshape(1, D)
  b3 = b3l.reshape(1, D)

  seg = _sc_segsum(N, D, E, gather=True)
  CW = 128
  seg_cnt = _sc_segsum(N, CW, E, gather=False)
  zeros_c = jnp.zeros((N, CW), jnp.float32)

  cnt = seg_cnt(src, dst, zeros_c, zeros_c)
  h1, xr1 = _tc_pre(x_all, W1l, W1r, block)
  s1 = seg(src, dst, h1, zeros)
  h2, xr2 = _tc_mid(s1, cnt, xr1, b1, W2l, W2r, block)
  s2 = seg(src, dst, h2, zeros)
  h3, xr3 = _tc_mid(s2, cnt, xr2, b2, W3l, W3r, block)
  s3 = seg(src, dst, h3, zeros)
  return _tc_post(s3, cnt, xr3, b3, block)


def kernel(edge_index, x_all, W1l, b1l, W1r, W2l, b2l, W2r, W3l, b3l, W3r):
  return _run(edge_index, x_all, W1l, b1l, W1r, W2l, b2l, W2r, W3l, b3l, W3r)
